# Initial kernel scaffold; baseline (speedup 1.0000x reference)
import jax
import jax.numpy as jnp
from jax.experimental import pallas as pl


def kernel(xyz, features, W1, b1, W2, b2):
    raise NotImplementedError("write your pallas kernel here")


# speedup vs baseline: 2.8404x; 2.8404x over previous
import functools

import jax
import jax.numpy as jnp
from jax import lax
from jax.experimental import pallas as pl
from jax.experimental.pallas import tpu as pltpu

B_SZ = 4
N_PTS = 4096
C_FEAT = 128
M_Q = 1024
NS = 32
RAD = 0.8
H1_D = 64
H2_D = 128


def _f1_body(feats_ref, w1f_ref, out_ref):
    f = feats_ref[0]
    w = w1f_ref[...]
    out_ref[0] = lax.dot_general(
        f, w, (((0,), (0,)), ((), ())), preferred_element_type=jnp.float32
    )


def _compute_f1(features, w1f):
    return pl.pallas_call(
        _f1_body,
        grid=(B_SZ,),
        in_specs=[
            pl.BlockSpec((1, C_FEAT, N_PTS), lambda b: (b, 0, 0)),
            pl.BlockSpec((C_FEAT, H1_D), lambda b: (0, 0)),
        ],
        out_specs=pl.BlockSpec((1, N_PTS, H1_D), lambda b: (b, 0, 0)),
        out_shape=jax.ShapeDtypeStruct((B_SZ, N_PTS, H1_D), jnp.float32),
    )(features, w1f)


def _fps_body(xt_ref, xyz_ref, nxyz_ref):
    X = xt_ref[:, 0, :]
    Y = xt_ref[:, 1, :]
    Z = xt_ref[:, 2, :]
    iota = lax.broadcasted_iota(jnp.int32, (B_SZ, N_PTS), 1)

    rows0 = [xyz_ref[b, pl.ds(0, 1), :] for b in range(B_SZ)]
    for b in range(B_SZ):
        nxyz_ref[b, pl.ds(0, 1), :] = rows0[b]
    last = jnp.concatenate(rows0, axis=0)
    lx0 = last[:, 0:1]
    ly0 = last[:, 1:2]
    lz0 = last[:, 2:3]
    dist0 = jnp.full((B_SZ, N_PTS), 1e10, dtype=jnp.float32)

    def body(i, carry):
        dist, lx, ly, lz = carry
        dx = X - lx
        dy = Y - ly
        dz = Z - lz
        d = (dx * dx + dy * dy) + dz * dz
        dist = jnp.minimum(dist, d)
        mx = jnp.max(dist, axis=1, keepdims=True)
        nxt = jnp.min(
            jnp.where(dist == mx, iota, N_PTS), axis=1, keepdims=True
        )
        rows = []
        for b in range(B_SZ):
            row = xyz_ref[b, pl.ds(nxt[b, 0], 1), :]
            nxyz_ref[b, pl.ds(i, 1), :] = row
            rows.append(row)
        nl = jnp.concatenate(rows, axis=0)
        return dist, nl[:, 0:1], nl[:, 1:2], nl[:, 2:3]

    lax.fori_loop(1, M_Q, body, (dist0, lx0, ly0, lz0))


def _run_fps(xt, xyz):
    return pl.pallas_call(
        _fps_body,
        out_shape=jax.ShapeDtypeStruct((B_SZ, M_Q, 3), jnp.float32),
    )(xt, xyz)


_TM = 128


def _bq_body(xt_ref, q_ref, idx_ref, flag_ref):
    r2 = jnp.float32(RAD * RAD)
    xr = xt_ref[0, 0:1, :]
    yr = xt_ref[0, 1:2, :]
    zr = xt_ref[0, 2:3, :]
    qx = q_ref[0, :, 0:1]
    qy = q_ref[0, :, 1:2]
    qz = q_ref[0, :, 2:3]
    dx = qx - xr
    dy = qy - yr
    dz = qz - zr
    d = (dx * dx + dy * dy) + dz * dz
    iota = lax.broadcasted_iota(jnp.int32, (_TM, N_PTS), 1)
    K = jnp.where(d < r2, iota, N_PTS)
    cur = K
    cols = []
    for _ in range(NS):
        v = jnp.min(cur, axis=1, keepdims=True)
        cols.append(v)
        cur = jnp.where(cur == v, N_PTS, cur)
    idxs = jnp.concatenate(cols, axis=1)
    first = idxs[:, 0:1]
    idxs = jnp.where(idxs < N_PTS, idxs, first)
    idxs = jnp.where(idxs < N_PTS, idxs, 0)
    idx_ref[0] = idxs
    flag_ref[0] = (first < N_PTS).astype(jnp.float32)


def _run_ball_query(xt, new_xyz):
    return pl.pallas_call(
        _bq_body,
        grid=(B_SZ, M_Q // _TM),
        in_specs=[
            pl.BlockSpec((1, 3, N_PTS), lambda b, m: (b, 0, 0)),
            pl.BlockSpec((1, _TM, 3), lambda b, m: (b, m, 0)),
        ],
        out_specs=[
            pl.BlockSpec((1, _TM, NS), lambda b, m: (b, m, 0)),
            pl.BlockSpec((1, _TM, 1), lambda b, m: (b, m, 0)),
        ],
        out_shape=[
            jax.ShapeDtypeStruct((B_SZ, M_Q, NS), jnp.int32),
            jax.ShapeDtypeStruct((B_SZ, M_Q, 1), jnp.float32),
        ],
    )(xt, new_xyz)


_TMD = 256


def _mlp_body(g1_ref, xr_ref, w1x_ref, b1_ref, w2_ref, b2_ref, flag_ref, out_ref):
    g = g1_ref[0].reshape(_TMD * NS, H1_D)
    xr = xr_ref[0].reshape(_TMD * NS, 3)
    a = g
    for c in range(3):
        a = a + xr[:, c : c + 1] * w1x_ref[c : c + 1, :]
    a = a + b1_ref[...]
    h = jnp.maximum(a, 0.0)
    h2 = lax.dot_general(
        h, w2_ref[...], (((1,), (0,)), ((), ())), preferred_element_type=jnp.float32
    )
    h2 = jnp.maximum(h2 + b2_ref[...], 0.0)
    pooled = jnp.max(h2.reshape(_TMD, NS, H2_D), axis=1)
    out_ref[0] = pooled * flag_ref[0]


def _run_mlp(g1, xr, w1x, b1, w2, b2, flag):
    return pl.pallas_call(
        _mlp_body,
        grid=(B_SZ, M_Q // _TMD),
        in_specs=[
            pl.BlockSpec((1, _TMD, NS, H1_D), lambda b, m: (b, m, 0, 0)),
            pl.BlockSpec((1, _TMD, NS, 3), lambda b, m: (b, m, 0, 0)),
            pl.BlockSpec((3, H1_D), lambda b, m: (0, 0)),
            pl.BlockSpec((1, H1_D), lambda b, m: (0, 0)),
            pl.BlockSpec((H1_D, H2_D), lambda b, m: (0, 0)),
            pl.BlockSpec((1, H2_D), lambda b, m: (0, 0)),
            pl.BlockSpec((1, _TMD, 1), lambda b, m: (b, m, 0)),
        ],
        out_specs=pl.BlockSpec((1, _TMD, H2_D), lambda b, m: (b, m, 0)),
        out_shape=jax.ShapeDtypeStruct((B_SZ, M_Q, H2_D), jnp.float32),
    )(g1, xr, w1x, b1, w2, b2, flag)


def kernel(xyz, features, W1, b1, W2, b2):
    xt = jnp.transpose(xyz, (0, 2, 1))
    f1 = _compute_f1(features, W1[3:])
    new_xyz = _run_fps(xt, xyz)
    idx, flag = _run_ball_query(xt, new_xyz)

    flat = idx.reshape(B_SZ, M_Q * NS, 1)
    g1 = jnp.take_along_axis(f1, flat, axis=1).reshape(B_SZ, M_Q, NS, H1_D)
    gxyz = jnp.take_along_axis(xyz, flat, axis=1).reshape(B_SZ, M_Q, NS, 3)
    xr = gxyz - new_xyz[:, :, None, :]

    pooled = _run_mlp(
        g1, xr, W1[:3], b1.reshape(1, H1_D), W2, b2.reshape(1, H2_D), flag
    )
    new_features = jnp.transpose(pooled, (0, 2, 1))
    return (new_xyz, new_features, None)
